# initial kernel scaffold (unmeasured)
import jax
import jax.numpy as jnp
from jax import lax
from jax.experimental import pallas as pl
from jax.experimental.pallas import tpu as pltpu

N_DEV = 4


def kernel(t, W):
    t = t.astype(jnp.bfloat16)
    W = W.astype(jnp.bfloat16)
    m_shard, k = t.shape
    n = W.shape[1]
    mc = m_shard // N_DEV

    def body(t_hbm, w_ref, out_hbm, L, R, ochunk,
             ldma_sem, odma_sems, rs_send, rs_recv, ag_send, ag_recv):
        my = lax.axis_index("i")
        left = lax.rem(my + N_DEV - 1, N_DEV)
        right = lax.rem(my + 1, N_DEV)

        c0 = lax.rem(my + N_DEV - 1, N_DEV)
        cp = pltpu.make_async_copy(
            t_hbm.at[pl.ds(c0 * mc, mc), :], L, ldma_sem)
        cp.start()

        barrier_sem = pltpu.get_barrier_semaphore()
        for nbr in (left, right):
            pl.semaphore_signal(barrier_sem, inc=1, device_id=(nbr,),
                                device_id_type=pl.DeviceIdType.MESH)
        pl.semaphore_wait(barrier_sem, 2)
        cp.wait()

        for h in range(N_DEV - 1):
            src = L if h == 0 else R.at[h - 1]
            rdma = pltpu.make_async_remote_copy(
                src_ref=src, dst_ref=R.at[h],
                send_sem=rs_send.at[h], recv_sem=rs_recv.at[h],
                device_id=(right,), device_id_type=pl.DeviceIdType.MESH)
            rdma.start()
            rdma.wait()
            ci = lax.rem(my + N_DEV + 2 - h, N_DEV)
            cp = pltpu.make_async_copy(
                t_hbm.at[pl.ds(ci * mc, mc), :], L, ldma_sem)
            cp.start()
            cp.wait()
            R[h] = R[h] + L[...]

        ochunk[...] = jnp.dot(
            R[N_DEV - 2], w_ref[...],
            preferred_element_type=jnp.float32).astype(jnp.bfloat16)

        out_cp = pltpu.make_async_copy(
            ochunk, out_hbm.at[pl.ds(my * mc, mc), :], odma_sems.at[N_DEV - 1])
        out_cp.start()
        pending = [out_cp]

        for h in range(N_DEV - 1):
            src = ochunk if h == 0 else R.at[h - 1]
            rdma = pltpu.make_async_remote_copy(
                src_ref=src, dst_ref=R.at[h],
                send_sem=ag_send.at[h], recv_sem=ag_recv.at[h],
                device_id=(right,), device_id_type=pl.DeviceIdType.MESH)
            rdma.start()
            rdma.wait()
            org = lax.rem(my + N_DEV - 1 - h, N_DEV)
            cp = pltpu.make_async_copy(
                R.at[h], out_hbm.at[pl.ds(org * mc, mc), :], odma_sems.at[h])
            cp.start()
            pending.append(cp)

        for cp in pending:
            cp.wait()

    return pl.pallas_call(
        body,
        out_shape=jax.ShapeDtypeStruct((m_shard, n), jnp.bfloat16),
        in_specs=[
            pl.BlockSpec(memory_space=pltpu.ANY),
            pl.BlockSpec(memory_space=pltpu.VMEM),
        ],
        out_specs=pl.BlockSpec(memory_space=pltpu.ANY),
        scratch_shapes=[
            pltpu.VMEM((mc, k), jnp.bfloat16),
            pltpu.VMEM((N_DEV - 1, mc, k), jnp.bfloat16),
            pltpu.VMEM((mc, n), jnp.bfloat16),
            pltpu.SemaphoreType.DMA,
            pltpu.SemaphoreType.DMA((N_DEV,)),
            pltpu.SemaphoreType.DMA((N_DEV - 1,)),
            pltpu.SemaphoreType.DMA((N_DEV - 1,)),
            pltpu.SemaphoreType.DMA((N_DEV - 1,)),
            pltpu.SemaphoreType.DMA((N_DEV - 1,)),
        ],
        compiler_params=pltpu.CompilerParams(collective_id=0),
    )(t, W)


# baseline (device time: 657914 ns/iter reference)
import jax
import jax.numpy as jnp
from jax import lax
from jax.experimental import pallas as pl
from jax.experimental.pallas import tpu as pltpu

N_DEV = 4


def kernel(t, W):
    t = t.astype(jnp.bfloat16)
    W = W.astype(jnp.bfloat16)
    m_shard, k = t.shape
    n = W.shape[1]
    mc = m_shard // N_DEV

    def body(t_hbm, w_ref, out_hbm, L, R, ochunk,
             ldma_sem, odma_sems, rs_send, rs_recv, ag_send, ag_recv):
        my = lax.axis_index("i")
        left = lax.rem(my + N_DEV - 1, N_DEV)
        right = lax.rem(my + 1, N_DEV)

        c0 = lax.rem(my + N_DEV - 1, N_DEV)
        cp = pltpu.make_async_copy(
            t_hbm.at[pl.ds(c0 * mc, mc), :], L, ldma_sem)
        cp.start()

        barrier_sem = pltpu.get_barrier_semaphore()
        for nbr in (left, right):
            pl.semaphore_signal(barrier_sem, inc=1, device_id=(nbr,),
                                device_id_type=pl.DeviceIdType.MESH)
        pl.semaphore_wait(barrier_sem, 2)
        cp.wait()

        for h in range(N_DEV - 1):
            src = L if h == 0 else R.at[h - 1]
            rdma = pltpu.make_async_remote_copy(
                src_ref=src, dst_ref=R.at[h],
                send_sem=rs_send.at[h], recv_sem=rs_recv.at[h],
                device_id=(right,), device_id_type=pl.DeviceIdType.MESH)
            rdma.start()
            rdma.wait()
            ci = lax.rem(my + N_DEV + 2 - h, N_DEV)
            cp = pltpu.make_async_copy(
                t_hbm.at[pl.ds(ci * mc, mc), :], L, ldma_sem)
            cp.start()
            cp.wait()
            R[h] = R[h] + L[...]

        ochunk[...] = jnp.dot(
            R[N_DEV - 2], w_ref[...],
            preferred_element_type=jnp.float32).astype(jnp.bfloat16)

        out_cp = pltpu.make_async_copy(
            ochunk, out_hbm.at[pl.ds(my * mc, mc), :], odma_sems.at[N_DEV - 1])
        out_cp.start()
        pending = [out_cp]

        for h in range(N_DEV - 1):
            src = ochunk if h == 0 else R.at[h - 1]
            rdma = pltpu.make_async_remote_copy(
                src_ref=src, dst_ref=R.at[h],
                send_sem=ag_send.at[h], recv_sem=ag_recv.at[h],
                device_id=(right,), device_id_type=pl.DeviceIdType.MESH)
            rdma.start()
            rdma.wait()
            org = lax.rem(my + N_DEV - 1 - h, N_DEV)
            cp = pltpu.make_async_copy(
                R.at[h], out_hbm.at[pl.ds(org * mc, mc), :], odma_sems.at[h])
            cp.start()
            pending.append(cp)

        for cp in pending:
            cp.wait()

    return pl.pallas_call(
        body,
        out_shape=jax.ShapeDtypeStruct((m_shard, n), jnp.bfloat16),
        in_specs=[
            pl.BlockSpec(memory_space=pl.ANY),
            pl.BlockSpec(memory_space=pltpu.VMEM),
        ],
        out_specs=pl.BlockSpec(memory_space=pl.ANY),
        scratch_shapes=[
            pltpu.VMEM((mc, k), jnp.bfloat16),
            pltpu.VMEM((N_DEV - 1, mc, k), jnp.bfloat16),
            pltpu.VMEM((mc, n), jnp.bfloat16),
            pltpu.SemaphoreType.DMA,
            pltpu.SemaphoreType.DMA((N_DEV,)),
            pltpu.SemaphoreType.DMA((N_DEV - 1,)),
            pltpu.SemaphoreType.DMA((N_DEV - 1,)),
            pltpu.SemaphoreType.DMA((N_DEV - 1,)),
            pltpu.SemaphoreType.DMA((N_DEV - 1,)),
        ],
        compiler_params=pltpu.CompilerParams(
            collective_id=0, vmem_limit_bytes=60 * 1024 * 1024),
    )(t, W)


# device time: 378744 ns/iter; 1.7371x vs baseline; 1.7371x over previous
import jax
import jax.numpy as jnp
from jax import lax
from jax.experimental import pallas as pl
from jax.experimental.pallas import tpu as pltpu

N_DEV = 4


def kernel(t, W):
    t = t.astype(jnp.bfloat16)
    W = W.astype(jnp.bfloat16)
    m_shard, k = t.shape
    n = W.shape[1]
    mc = m_shard // N_DEV
    kh = k // 2
    nh = n // 2

    def body(t_hbm, w_ref, out_hbm, SA, SB, RA, RB, OA, OB,
             sdma, odma, rsa_s, rsa_r, rsb_s, rsb_r, aga_s, aga_r,
             agb_s, agb_r):
        my = lax.axis_index("i")
        left = lax.rem(my + N_DEV - 1, N_DEV)
        right = lax.rem(my + 1, N_DEV)

        def row(c):
            return pl.ds(lax.rem(c, N_DEV) * mc, mc)

        ld_oa = pltpu.make_async_copy(
            t_hbm.at[row(my + 3), pl.ds(0, kh)], OA, sdma.at[0])
        ld_ob = pltpu.make_async_copy(
            t_hbm.at[row(my + 1), pl.ds(kh, kh)], OB, sdma.at[1])
        ld_oa.start()
        ld_ob.start()
        ld_sa = pltpu.make_async_copy(
            t_hbm.at[row(my + 2), pl.ds(0, kh)], SA, sdma.at[2])
        ld_sb = pltpu.make_async_copy(
            t_hbm.at[row(my + 2), pl.ds(kh, kh)], SB, sdma.at[3])
        ld_sa.start()
        ld_sb.start()

        barrier_sem = pltpu.get_barrier_semaphore()
        for nbr in (left, right):
            pl.semaphore_signal(barrier_sem, inc=1, device_id=(nbr,),
                                device_id_type=pl.DeviceIdType.MESH)
        pl.semaphore_wait(barrier_sem, 2)
        ld_oa.wait()
        ld_ob.wait()

        for h in range(N_DEV - 1):
            rdma_a = pltpu.make_async_remote_copy(
                src_ref=OA if h == 0 else RA.at[h - 1], dst_ref=RA.at[h],
                send_sem=rsa_s.at[h], recv_sem=rsa_r.at[h],
                device_id=(right,), device_id_type=pl.DeviceIdType.MESH)
            rdma_b = pltpu.make_async_remote_copy(
                src_ref=OB if h == 0 else RB.at[h - 1], dst_ref=RB.at[h],
                send_sem=rsb_s.at[h], recv_sem=rsb_r.at[h],
                device_id=(left,), device_id_type=pl.DeviceIdType.MESH)
            rdma_a.start()
            rdma_b.start()
            rdma_a.wait()
            rdma_b.wait()
            ld_sa.wait()
            ld_sb.wait()
            RA[h] = RA[h] + SA[...]
            RB[h] = RB[h] + SB[...]
            if h < N_DEV - 2:
                ld_sa = pltpu.make_async_copy(
                    t_hbm.at[row(my + 2 - (h + 1)), pl.ds(0, kh)],
                    SA, sdma.at[2])
                ld_sb = pltpu.make_async_copy(
                    t_hbm.at[row(my + 2 + (h + 1)), pl.ds(kh, kh)],
                    SB, sdma.at[3])
                ld_sa.start()
                ld_sb.start()

        for r in (0, mc // 2):
            rs = pl.ds(r, mc // 2)
            ca = RA[N_DEV - 2, rs, :]
            cb = RB[N_DEV - 2, rs, :]
            OA[rs, :] = (
                jnp.dot(ca, w_ref[pl.ds(0, kh), pl.ds(0, nh)],
                        preferred_element_type=jnp.float32)
                + jnp.dot(cb, w_ref[pl.ds(kh, kh), pl.ds(0, nh)],
                          preferred_element_type=jnp.float32)
            ).astype(jnp.bfloat16)
            OB[rs, :] = (
                jnp.dot(ca, w_ref[pl.ds(0, kh), pl.ds(nh, nh)],
                        preferred_element_type=jnp.float32)
                + jnp.dot(cb, w_ref[pl.ds(kh, kh), pl.ds(nh, nh)],
                          preferred_element_type=jnp.float32)
            ).astype(jnp.bfloat16)

        st_oa = pltpu.make_async_copy(
            OA, out_hbm.at[row(my), pl.ds(0, nh)], odma.at[6])
        st_ob = pltpu.make_async_copy(
            OB, out_hbm.at[row(my), pl.ds(nh, nh)], odma.at[7])
        st_oa.start()
        st_ob.start()
        pending = [st_oa, st_ob]

        for h in range(N_DEV - 1):
            rdma_a = pltpu.make_async_remote_copy(
                src_ref=OA if h == 0 else RA.at[h - 1], dst_ref=RA.at[h],
                send_sem=aga_s.at[h], recv_sem=aga_r.at[h],
                device_id=(right,), device_id_type=pl.DeviceIdType.MESH)
            rdma_b = pltpu.make_async_remote_copy(
                src_ref=OB if h == 0 else RB.at[h - 1], dst_ref=RB.at[h],
                send_sem=agb_s.at[h], recv_sem=agb_r.at[h],
                device_id=(left,), device_id_type=pl.DeviceIdType.MESH)
            rdma_a.start()
            rdma_b.start()
            rdma_a.wait()
            rdma_b.wait()
            st_a = pltpu.make_async_copy(
                RA.at[h], out_hbm.at[row(my - 1 - h + N_DEV), pl.ds(0, nh)],
                odma.at[2 * h])
            st_b = pltpu.make_async_copy(
                RB.at[h], out_hbm.at[row(my + 1 + h), pl.ds(nh, nh)],
                odma.at[2 * h + 1])
            st_a.start()
            st_b.start()
            pending.append(st_a)
            pending.append(st_b)

        for cp in pending:
            cp.wait()

    assert kh == nh

    return pl.pallas_call(
        body,
        out_shape=jax.ShapeDtypeStruct((m_shard, n), jnp.bfloat16),
        in_specs=[
            pl.BlockSpec(memory_space=pl.ANY),
            pl.BlockSpec(memory_space=pltpu.VMEM),
        ],
        out_specs=pl.BlockSpec(memory_space=pl.ANY),
        scratch_shapes=[
            pltpu.VMEM((mc, kh), jnp.bfloat16),
            pltpu.VMEM((mc, kh), jnp.bfloat16),
            pltpu.VMEM((N_DEV - 1, mc, kh), jnp.bfloat16),
            pltpu.VMEM((N_DEV - 1, mc, kh), jnp.bfloat16),
            pltpu.VMEM((mc, nh), jnp.bfloat16),
            pltpu.VMEM((mc, nh), jnp.bfloat16),
            pltpu.SemaphoreType.DMA((4,)),
            pltpu.SemaphoreType.DMA((8,)),
            pltpu.SemaphoreType.DMA((N_DEV - 1,)),
            pltpu.SemaphoreType.DMA((N_DEV - 1,)),
            pltpu.SemaphoreType.DMA((N_DEV - 1,)),
            pltpu.SemaphoreType.DMA((N_DEV - 1,)),
            pltpu.SemaphoreType.DMA((N_DEV - 1,)),
            pltpu.SemaphoreType.DMA((N_DEV - 1,)),
            pltpu.SemaphoreType.DMA((N_DEV - 1,)),
            pltpu.SemaphoreType.DMA((N_DEV - 1,)),
        ],
        compiler_params=pltpu.CompilerParams(
            collective_id=0, vmem_limit_bytes=63 * 1024 * 1024),
    )(t, W)


# device time: 349199 ns/iter; 1.8841x vs baseline; 1.0846x over previous
import jax
import jax.numpy as jnp
from jax import lax
from jax.experimental import pallas as pl
from jax.experimental.pallas import tpu as pltpu

N_DEV = 4


def kernel(t, W):
    W = W.astype(jnp.bfloat16)
    m_shard, k = t.shape
    n = W.shape[1]
    mc = m_shard // N_DEV
    kh = k // 2
    nh = n // 2

    def body(t_hbm, w_ref, out_hbm, SA, SB, RA, RB, OA, OB,
             sdma, odma, rsa_s, rsa_r, rsb_s, rsb_r, aga_s, aga_r,
             agb_s, agb_r):
        my = lax.axis_index("i")
        left = lax.rem(my + N_DEV - 1, N_DEV)
        right = lax.rem(my + 1, N_DEV)

        def row(c):
            return pl.ds(lax.rem(c, N_DEV) * mc, mc)

        ld_sa = pltpu.make_async_copy(
            t_hbm.at[row(my + 3), pl.ds(0, kh)], SA, sdma.at[0])
        ld_sb = pltpu.make_async_copy(
            t_hbm.at[row(my + 1), pl.ds(kh, kh)], SB, sdma.at[1])
        ld_sa.start()
        ld_sb.start()

        barrier_sem = pltpu.get_barrier_semaphore()
        for nbr in (left, right):
            pl.semaphore_signal(barrier_sem, inc=1, device_id=(nbr,),
                                device_id_type=pl.DeviceIdType.MESH)

        ld_sa.wait()
        OA[...] = SA[...].astype(jnp.bfloat16)
        ld_sb.wait()
        OB[...] = SB[...].astype(jnp.bfloat16)
        ld_sa = pltpu.make_async_copy(
            t_hbm.at[row(my + 2), pl.ds(0, kh)], SA, sdma.at[2])
        ld_sb = pltpu.make_async_copy(
            t_hbm.at[row(my + 2), pl.ds(kh, kh)], SB, sdma.at[3])
        ld_sa.start()
        ld_sb.start()

        pl.semaphore_wait(barrier_sem, 2)

        for h in range(N_DEV - 1):
            s = h % 2
            rdma_a = pltpu.make_async_remote_copy(
                src_ref=OA if h == 0 else RA.at[(h - 1) % 2],
                dst_ref=RA.at[s],
                send_sem=rsa_s.at[h], recv_sem=rsa_r.at[h],
                device_id=(right,), device_id_type=pl.DeviceIdType.MESH)
            rdma_b = pltpu.make_async_remote_copy(
                src_ref=OB if h == 0 else RB.at[(h - 1) % 2],
                dst_ref=RB.at[s],
                send_sem=rsb_s.at[h], recv_sem=rsb_r.at[h],
                device_id=(left,), device_id_type=pl.DeviceIdType.MESH)
            rdma_a.start()
            rdma_b.start()
            rdma_a.wait()
            rdma_b.wait()
            ld_sa.wait()
            ld_sb.wait()
            RA[s] = RA[s] + SA[...].astype(jnp.bfloat16)
            RB[s] = RB[s] + SB[...].astype(jnp.bfloat16)
            if h < N_DEV - 2:
                ld_sa = pltpu.make_async_copy(
                    t_hbm.at[row(my + 2 - (h + 1)), pl.ds(0, kh)],
                    SA, sdma.at[2])
                ld_sb = pltpu.make_async_copy(
                    t_hbm.at[row(my + 2 + (h + 1)), pl.ds(kh, kh)],
                    SB, sdma.at[3])
                ld_sa.start()
                ld_sb.start()

        for r in (0, mc // 2):
            rs = pl.ds(r, mc // 2)
            OA[rs, :] = (
                jnp.dot(RA[0, rs, :], w_ref[pl.ds(0, kh), pl.ds(0, nh)],
                        preferred_element_type=jnp.float32)
                + jnp.dot(RB[0, rs, :], w_ref[pl.ds(kh, kh), pl.ds(0, nh)],
                          preferred_element_type=jnp.float32)
            ).astype(jnp.bfloat16)
        ag_a = pltpu.make_async_remote_copy(
            src_ref=OA, dst_ref=RA.at[1],
            send_sem=aga_s.at[0], recv_sem=aga_r.at[0],
            device_id=(right,), device_id_type=pl.DeviceIdType.MESH)
        ag_a.start()
        st_oa = pltpu.make_async_copy(
            OA, out_hbm.at[row(my), pl.ds(0, nh)], odma.at[6])
        st_oa.start()

        for r in (0, mc // 2):
            rs = pl.ds(r, mc // 2)
            OB[rs, :] = (
                jnp.dot(RA[0, rs, :], w_ref[pl.ds(0, kh), pl.ds(nh, nh)],
                        preferred_element_type=jnp.float32)
                + jnp.dot(RB[0, rs, :], w_ref[pl.ds(kh, kh), pl.ds(nh, nh)],
                          preferred_element_type=jnp.float32)
            ).astype(jnp.bfloat16)
        ag_b = pltpu.make_async_remote_copy(
            src_ref=OB, dst_ref=RB.at[1],
            send_sem=agb_s.at[0], recv_sem=agb_r.at[0],
            device_id=(left,), device_id_type=pl.DeviceIdType.MESH)
        ag_b.start()
        st_ob = pltpu.make_async_copy(
            OB, out_hbm.at[row(my), pl.ds(nh, nh)], odma.at[7])
        st_ob.start()
        pending = [st_oa, st_ob]

        for h in range(N_DEV - 1):
            s = (h + 1) % 2
            if h == 0:
                rdma_a, rdma_b = ag_a, ag_b
            else:
                rdma_a = pltpu.make_async_remote_copy(
                    src_ref=RA.at[h % 2], dst_ref=RA.at[s],
                    send_sem=aga_s.at[h], recv_sem=aga_r.at[h],
                    device_id=(right,), device_id_type=pl.DeviceIdType.MESH)
                rdma_b = pltpu.make_async_remote_copy(
                    src_ref=RB.at[h % 2], dst_ref=RB.at[s],
                    send_sem=agb_s.at[h], recv_sem=agb_r.at[h],
                    device_id=(left,), device_id_type=pl.DeviceIdType.MESH)
                rdma_a.start()
                rdma_b.start()
            rdma_a.wait()
            rdma_b.wait()
            st_a = pltpu.make_async_copy(
                RA.at[s], out_hbm.at[row(my - 1 - h + N_DEV), pl.ds(0, nh)],
                odma.at[2 * h])
            st_b = pltpu.make_async_copy(
                RB.at[s], out_hbm.at[row(my + 1 + h), pl.ds(nh, nh)],
                odma.at[2 * h + 1])
            st_a.start()
            st_b.start()
            pending.append(st_a)
            pending.append(st_b)

        for cp in pending:
            cp.wait()

    assert kh == nh

    return pl.pallas_call(
        body,
        out_shape=jax.ShapeDtypeStruct((m_shard, n), jnp.bfloat16),
        in_specs=[
            pl.BlockSpec(memory_space=pl.ANY),
            pl.BlockSpec(memory_space=pltpu.VMEM),
        ],
        out_specs=pl.BlockSpec(memory_space=pl.ANY),
        scratch_shapes=[
            pltpu.VMEM((mc, kh), jnp.float32),
            pltpu.VMEM((mc, kh), jnp.float32),
            pltpu.VMEM((2, mc, kh), jnp.bfloat16),
            pltpu.VMEM((2, mc, kh), jnp.bfloat16),
            pltpu.VMEM((mc, nh), jnp.bfloat16),
            pltpu.VMEM((mc, nh), jnp.bfloat16),
            pltpu.SemaphoreType.DMA((4,)),
            pltpu.SemaphoreType.DMA((8,)),
            pltpu.SemaphoreType.DMA((N_DEV - 1,)),
            pltpu.SemaphoreType.DMA((N_DEV - 1,)),
            pltpu.SemaphoreType.DMA((N_DEV - 1,)),
            pltpu.SemaphoreType.DMA((N_DEV - 1,)),
            pltpu.SemaphoreType.DMA((N_DEV - 1,)),
            pltpu.SemaphoreType.DMA((N_DEV - 1,)),
            pltpu.SemaphoreType.DMA((N_DEV - 1,)),
            pltpu.SemaphoreType.DMA((N_DEV - 1,)),
        ],
        compiler_params=pltpu.CompilerParams(
            collective_id=0, vmem_limit_bytes=63 * 1024 * 1024),
    )(t, W)


# device time: 342710 ns/iter; 1.9197x vs baseline; 1.0189x over previous
import jax
import jax.numpy as jnp
from jax import lax
from jax.experimental import pallas as pl
from jax.experimental.pallas import tpu as pltpu

N_DEV = 4


def kernel(t, W):
    W = W.astype(jnp.bfloat16)
    m_shard, k = t.shape
    n = W.shape[1]
    mc = m_shard // N_DEV
    kh = k // 2
    nh = n // 2

    def body(t_hbm, w_ref, out_hbm, SA, SB, RA, RB, OA, OB,
             sdma, odma, rsa_s, rsa_r, rsb_s, rsb_r, aga_s, aga_r,
             agb_s, agb_r):
        my = lax.axis_index("i")
        left = lax.rem(my + N_DEV - 1, N_DEV)
        right = lax.rem(my + 1, N_DEV)

        def row(c):
            return pl.ds(lax.rem(c, N_DEV) * mc, mc)

        ld_sa = pltpu.make_async_copy(
            t_hbm.at[row(my + 3), pl.ds(0, kh)], SA, sdma.at[0])
        ld_sb = pltpu.make_async_copy(
            t_hbm.at[row(my + 1), pl.ds(kh, kh)], SB, sdma.at[1])
        ld_sa.start()
        ld_sb.start()

        barrier_sem = pltpu.get_barrier_semaphore()
        for nbr in (left, right):
            pl.semaphore_signal(barrier_sem, inc=1, device_id=(nbr,),
                                device_id_type=pl.DeviceIdType.MESH)

        ld_sa.wait()
        OA[...] = SA[...].astype(jnp.bfloat16)
        ld_sb.wait()
        OB[...] = SB[...].astype(jnp.bfloat16)
        ld_sa = pltpu.make_async_copy(
            t_hbm.at[row(my + 2), pl.ds(0, kh)], SA, sdma.at[2])
        ld_sb = pltpu.make_async_copy(
            t_hbm.at[row(my + 2), pl.ds(kh, kh)], SB, sdma.at[3])
        ld_sa.start()
        ld_sb.start()

        pl.semaphore_wait(barrier_sem, 2)

        mch = mc // 2
        rs_descs = {}
        for h in range(N_DEV - 1):
            s = h % 2
            for j in range(2):
                rows = pl.ds(j * mch, mch)
                rs_descs["a", h, j] = pltpu.make_async_remote_copy(
                    src_ref=(OA.at[rows, :] if h == 0
                             else RA.at[(h - 1) % 2, rows, :]),
                    dst_ref=RA.at[s, rows, :],
                    send_sem=rsa_s.at[h, j], recv_sem=rsa_r.at[h, j],
                    device_id=(right,), device_id_type=pl.DeviceIdType.MESH)
                rs_descs["b", h, j] = pltpu.make_async_remote_copy(
                    src_ref=(OB.at[rows, :] if h == 0
                             else RB.at[(h - 1) % 2, rows, :]),
                    dst_ref=RB.at[s, rows, :],
                    send_sem=rsb_s.at[h, j], recv_sem=rsb_r.at[h, j],
                    device_id=(left,), device_id_type=pl.DeviceIdType.MESH)

        for j in range(2):
            rs_descs["a", 0, j].start()
            rs_descs["b", 0, j].start()
        for h in range(N_DEV - 1):
            s = h % 2
            ld_sa.wait()
            ld_sb.wait()
            for j in range(2):
                rows = pl.ds(j * mch, mch)
                rs_descs["a", h, j].wait()
                RA[s, rows, :] = (RA[s, rows, :]
                                  + SA[rows, :].astype(jnp.bfloat16))
                if h < N_DEV - 2:
                    rs_descs["a", h + 1, j].start()
                rs_descs["b", h, j].wait()
                RB[s, rows, :] = (RB[s, rows, :]
                                  + SB[rows, :].astype(jnp.bfloat16))
                if h < N_DEV - 2:
                    rs_descs["b", h + 1, j].start()
            if h < N_DEV - 2:
                ld_sa = pltpu.make_async_copy(
                    t_hbm.at[row(my + 2 - (h + 1)), pl.ds(0, kh)],
                    SA, sdma.at[2])
                ld_sb = pltpu.make_async_copy(
                    t_hbm.at[row(my + 2 + (h + 1)), pl.ds(kh, kh)],
                    SB, sdma.at[3])
                ld_sa.start()
                ld_sb.start()

        for r in (0, mc // 2):
            rs = pl.ds(r, mc // 2)
            OA[rs, :] = (
                jnp.dot(RA[0, rs, :], w_ref[pl.ds(0, kh), pl.ds(0, nh)],
                        preferred_element_type=jnp.float32)
                + jnp.dot(RB[0, rs, :], w_ref[pl.ds(kh, kh), pl.ds(0, nh)],
                          preferred_element_type=jnp.float32)
            ).astype(jnp.bfloat16)
        ag_a = pltpu.make_async_remote_copy(
            src_ref=OA, dst_ref=RA.at[1],
            send_sem=aga_s.at[0], recv_sem=aga_r.at[0],
            device_id=(right,), device_id_type=pl.DeviceIdType.MESH)
        ag_a.start()
        st_oa = pltpu.make_async_copy(
            OA, out_hbm.at[row(my), pl.ds(0, nh)], odma.at[6])
        st_oa.start()

        for r in (0, mc // 2):
            rs = pl.ds(r, mc // 2)
            OB[rs, :] = (
                jnp.dot(RA[0, rs, :], w_ref[pl.ds(0, kh), pl.ds(nh, nh)],
                        preferred_element_type=jnp.float32)
                + jnp.dot(RB[0, rs, :], w_ref[pl.ds(kh, kh), pl.ds(nh, nh)],
                          preferred_element_type=jnp.float32)
            ).astype(jnp.bfloat16)
        ag_b = pltpu.make_async_remote_copy(
            src_ref=OB, dst_ref=RB.at[1],
            send_sem=agb_s.at[0], recv_sem=agb_r.at[0],
            device_id=(left,), device_id_type=pl.DeviceIdType.MESH)
        ag_b.start()
        st_ob = pltpu.make_async_copy(
            OB, out_hbm.at[row(my), pl.ds(nh, nh)], odma.at[7])
        st_ob.start()
        pending = [st_oa, st_ob]

        for h in range(N_DEV - 1):
            s = (h + 1) % 2
            if h == 0:
                rdma_a, rdma_b = ag_a, ag_b
            else:
                rdma_a = pltpu.make_async_remote_copy(
                    src_ref=RA.at[h % 2], dst_ref=RA.at[s],
                    send_sem=aga_s.at[h], recv_sem=aga_r.at[h],
                    device_id=(right,), device_id_type=pl.DeviceIdType.MESH)
                rdma_b = pltpu.make_async_remote_copy(
                    src_ref=RB.at[h % 2], dst_ref=RB.at[s],
                    send_sem=agb_s.at[h], recv_sem=agb_r.at[h],
                    device_id=(left,), device_id_type=pl.DeviceIdType.MESH)
                rdma_a.start()
                rdma_b.start()
            rdma_a.wait()
            rdma_b.wait()
            st_a = pltpu.make_async_copy(
                RA.at[s], out_hbm.at[row(my - 1 - h + N_DEV), pl.ds(0, nh)],
                odma.at[2 * h])
            st_b = pltpu.make_async_copy(
                RB.at[s], out_hbm.at[row(my + 1 + h), pl.ds(nh, nh)],
                odma.at[2 * h + 1])
            st_a.start()
            st_b.start()
            pending.append(st_a)
            pending.append(st_b)

        for cp in pending:
            cp.wait()

    assert kh == nh

    return pl.pallas_call(
        body,
        out_shape=jax.ShapeDtypeStruct((m_shard, n), jnp.bfloat16),
        in_specs=[
            pl.BlockSpec(memory_space=pl.ANY),
            pl.BlockSpec(memory_space=pltpu.VMEM),
        ],
        out_specs=pl.BlockSpec(memory_space=pl.ANY),
        scratch_shapes=[
            pltpu.VMEM((mc, kh), jnp.float32),
            pltpu.VMEM((mc, kh), jnp.float32),
            pltpu.VMEM((2, mc, kh), jnp.bfloat16),
            pltpu.VMEM((2, mc, kh), jnp.bfloat16),
            pltpu.VMEM((mc, nh), jnp.bfloat16),
            pltpu.VMEM((mc, nh), jnp.bfloat16),
            pltpu.SemaphoreType.DMA((4,)),
            pltpu.SemaphoreType.DMA((8,)),
            pltpu.SemaphoreType.DMA((N_DEV - 1, 2)),
            pltpu.SemaphoreType.DMA((N_DEV - 1, 2)),
            pltpu.SemaphoreType.DMA((N_DEV - 1, 2)),
            pltpu.SemaphoreType.DMA((N_DEV - 1, 2)),
            pltpu.SemaphoreType.DMA((N_DEV - 1,)),
            pltpu.SemaphoreType.DMA((N_DEV - 1,)),
            pltpu.SemaphoreType.DMA((N_DEV - 1,)),
            pltpu.SemaphoreType.DMA((N_DEV - 1,)),
        ],
        compiler_params=pltpu.CompilerParams(
            collective_id=0, vmem_limit_bytes=63 * 1024 * 1024),
    )(t, W)


# device time: 316740 ns/iter; 2.0771x vs baseline; 1.0820x over previous
import jax
import jax.numpy as jnp
from jax import lax
from jax.experimental import pallas as pl
from jax.experimental.pallas import tpu as pltpu

N_DEV = 4


def kernel(t, W):
    W = W.astype(jnp.bfloat16)
    m_shard, k = t.shape
    n = W.shape[1]
    mc = m_shard // N_DEV
    kh = k // 2
    nh = n // 2
    mch = mc // 2

    def body(t_hbm, w_ref, out_hbm, SA, SB, RA, RB, OA, OB,
             sdma, odma, rsa_s, rsa_r, rsb_s, rsb_r, aga_s, aga_r,
             agb_s, agb_r):
        my = lax.axis_index("i")
        left = lax.rem(my + N_DEV - 1, N_DEV)
        right = lax.rem(my + 1, N_DEV)

        def row(c):
            return pl.ds(lax.rem(c, N_DEV) * mc, mc)

        def rowsub(c, j):
            return pl.ds(lax.rem(c, N_DEV) * mc + j * mch, mch)

        ld_sa = pltpu.make_async_copy(
            t_hbm.at[row(my + 3), pl.ds(0, kh)], SA, sdma.at[0])
        ld_sb = pltpu.make_async_copy(
            t_hbm.at[row(my + 1), pl.ds(kh, kh)], SB, sdma.at[1])
        ld_sa.start()
        ld_sb.start()

        barrier_sem = pltpu.get_barrier_semaphore()
        for nbr in (left, right):
            pl.semaphore_signal(barrier_sem, inc=1, device_id=(nbr,),
                                device_id_type=pl.DeviceIdType.MESH)

        ld_sa.wait()
        OA[...] = SA[...].astype(jnp.bfloat16)
        ld_sb.wait()
        OB[...] = SB[...].astype(jnp.bfloat16)
        ld_sa = pltpu.make_async_copy(
            t_hbm.at[row(my + 2), pl.ds(0, kh)], SA, sdma.at[2])
        ld_sb = pltpu.make_async_copy(
            t_hbm.at[row(my + 2), pl.ds(kh, kh)], SB, sdma.at[3])
        ld_sa.start()
        ld_sb.start()

        pl.semaphore_wait(barrier_sem, 2)

        rs = {}
        ag = {}
        for h in range(N_DEV - 1):
            for j in range(2):
                rows = pl.ds(j * mch, mch)
                rs["a", h, j] = pltpu.make_async_remote_copy(
                    src_ref=(OA.at[rows, :] if h == 0
                             else RA.at[(h - 1) % 2, rows, :]),
                    dst_ref=RA.at[h % 2, rows, :],
                    send_sem=rsa_s.at[h, j], recv_sem=rsa_r.at[h, j],
                    device_id=(right,), device_id_type=pl.DeviceIdType.MESH)
                rs["b", h, j] = pltpu.make_async_remote_copy(
                    src_ref=(OB.at[rows, :] if h == 0
                             else RB.at[(h - 1) % 2, rows, :]),
                    dst_ref=RB.at[h % 2, rows, :],
                    send_sem=rsb_s.at[h, j], recv_sem=rsb_r.at[h, j],
                    device_id=(left,), device_id_type=pl.DeviceIdType.MESH)
                ag["a", h, j] = pltpu.make_async_remote_copy(
                    src_ref=(OA.at[rows, :] if h == 0
                             else RA.at[h % 2, rows, :]),
                    dst_ref=RA.at[(h + 1) % 2, rows, :],
                    send_sem=aga_s.at[h, j], recv_sem=aga_r.at[h, j],
                    device_id=(right,), device_id_type=pl.DeviceIdType.MESH)
                ag["b", h, j] = pltpu.make_async_remote_copy(
                    src_ref=(OB.at[rows, :] if h == 0
                             else RB.at[h % 2, rows, :]),
                    dst_ref=RB.at[(h + 1) % 2, rows, :],
                    send_sem=agb_s.at[h, j], recv_sem=agb_r.at[h, j],
                    device_id=(left,), device_id_type=pl.DeviceIdType.MESH)

        def matmul_rows(dst, j, c0, c1):
            rows = pl.ds(j * mch, mch)
            dst[rows, :] = (
                jnp.dot(RA[0, rows, :], w_ref[pl.ds(0, kh), pl.ds(c0, nh)],
                        preferred_element_type=jnp.float32)
                + jnp.dot(RB[0, rows, :], w_ref[pl.ds(kh, kh), pl.ds(c0, nh)],
                          preferred_element_type=jnp.float32)
            ).astype(jnp.bfloat16)

        pending = []

        for j in range(2):
            rs["a", 0, j].start()
            rs["b", 0, j].start()
        for h in range(N_DEV - 1):
            s = h % 2
            last = h == N_DEV - 2
            ld_sa.wait()
            ld_sb.wait()
            for j in range(2):
                rows = pl.ds(j * mch, mch)
                rs["a", h, j].wait()
                RA[s, rows, :] = (RA[s, rows, :]
                                  + SA[rows, :].astype(jnp.bfloat16))
                if not last:
                    rs["a", h + 1, j].start()
                rs["b", h, j].wait()
                RB[s, rows, :] = (RB[s, rows, :]
                                  + SB[rows, :].astype(jnp.bfloat16))
                if not last:
                    rs["b", h + 1, j].start()
                if last:
                    matmul_rows(OA, j, 0, nh)
                    ag["a", 0, j].start()
                    matmul_rows(OB, j, nh, n)
                    ag["b", 0, j].start()
                    st_a = pltpu.make_async_copy(
                        OA.at[rows, :], out_hbm.at[rowsub(my, j), pl.ds(0, nh)],
                        odma.at[12 + 2 * j])
                    st_b = pltpu.make_async_copy(
                        OB.at[rows, :], out_hbm.at[rowsub(my, j), pl.ds(nh, nh)],
                        odma.at[13 + 2 * j])
                    st_a.start()
                    st_b.start()
                    pending.append(st_a)
                    pending.append(st_b)
            if h < N_DEV - 2:
                ld_sa = pltpu.make_async_copy(
                    t_hbm.at[row(my + 2 - (h + 1)), pl.ds(0, kh)],
                    SA, sdma.at[2])
                ld_sb = pltpu.make_async_copy(
                    t_hbm.at[row(my + 2 + (h + 1)), pl.ds(kh, kh)],
                    SB, sdma.at[3])
                ld_sa.start()
                ld_sb.start()

        for h in range(N_DEV - 1):
            recv_slot = (h + 1) % 2
            for j in range(2):
                rows = pl.ds(j * mch, mch)
                ag["a", h, j].wait()
                if h < N_DEV - 2:
                    ag["a", h + 1, j].start()
                st_a = pltpu.make_async_copy(
                    RA.at[recv_slot, rows, :],
                    out_hbm.at[rowsub(my - 1 - h + N_DEV, j), pl.ds(0, nh)],
                    odma.at[4 * h + 2 * j])
                st_a.start()
                pending.append(st_a)
                ag["b", h, j].wait()
                if h < N_DEV - 2:
                    ag["b", h + 1, j].start()
                st_b = pltpu.make_async_copy(
                    RB.at[recv_slot, rows, :],
                    out_hbm.at[rowsub(my + 1 + h, j), pl.ds(nh, nh)],
                    odma.at[4 * h + 2 * j + 1])
                st_b.start()
                pending.append(st_b)

        for cp in pending:
            cp.wait()

    assert kh == nh

    return pl.pallas_call(
        body,
        out_shape=jax.ShapeDtypeStruct((m_shard, n), jnp.bfloat16),
        in_specs=[
            pl.BlockSpec(memory_space=pl.ANY),
            pl.BlockSpec(memory_space=pltpu.VMEM),
        ],
        out_specs=pl.BlockSpec(memory_space=pl.ANY),
        scratch_shapes=[
            pltpu.VMEM((mc, kh), jnp.float32),
            pltpu.VMEM((mc, kh), jnp.float32),
            pltpu.VMEM((2, mc, kh), jnp.bfloat16),
            pltpu.VMEM((2, mc, kh), jnp.bfloat16),
            pltpu.VMEM((mc, nh), jnp.bfloat16),
            pltpu.VMEM((mc, nh), jnp.bfloat16),
            pltpu.SemaphoreType.DMA((4,)),
            pltpu.SemaphoreType.DMA((16,)),
            pltpu.SemaphoreType.DMA((N_DEV - 1, 2)),
            pltpu.SemaphoreType.DMA((N_DEV - 1, 2)),
            pltpu.SemaphoreType.DMA((N_DEV - 1, 2)),
            pltpu.SemaphoreType.DMA((N_DEV - 1, 2)),
            pltpu.SemaphoreType.DMA((N_DEV - 1, 2)),
            pltpu.SemaphoreType.DMA((N_DEV - 1, 2)),
            pltpu.SemaphoreType.DMA((N_DEV - 1, 2)),
            pltpu.SemaphoreType.DMA((N_DEV - 1, 2)),
        ],
        compiler_params=pltpu.CompilerParams(
            collective_id=0, vmem_limit_bytes=63 * 1024 * 1024),
    )(t, W)


# device time: 314332 ns/iter; 2.0931x vs baseline; 1.0077x over previous
import jax
import jax.numpy as jnp
from jax import lax
from jax.experimental import pallas as pl
from jax.experimental.pallas import tpu as pltpu

N_DEV = 4


def kernel(t, W):
    W = W.astype(jnp.bfloat16)
    m_shard, k = t.shape
    n = W.shape[1]
    mc = m_shard // N_DEV
    kh = k // 2
    nh = n // 2
    mch = mc // 2

    def body(t_hbm, w_ref, out_hbm, SA, SB, RA, RB, OA, OB,
             sdma, odma, rsa_s, rsa_r, rsb_s, rsb_r, aga_s, aga_r,
             agb_s, agb_r):
        my = lax.axis_index("i")
        left = lax.rem(my + N_DEV - 1, N_DEV)
        right = lax.rem(my + 1, N_DEV)

        def row(c):
            return pl.ds(lax.rem(c, N_DEV) * mc, mc)

        def rowsub(c, j):
            return pl.ds(lax.rem(c, N_DEV) * mc + j * mch, mch)

        ld0 = []
        for j in range(2):
            rows = pl.ds(j * mch, mch)
            la = pltpu.make_async_copy(
                t_hbm.at[rowsub(my + 3, j), pl.ds(0, kh)],
                SA.at[rows, :], sdma.at[2 * j])
            lb = pltpu.make_async_copy(
                t_hbm.at[rowsub(my + 1, j), pl.ds(kh, kh)],
                SB.at[rows, :], sdma.at[2 * j + 1])
            la.start()
            lb.start()
            ld0.append((la, lb))

        barrier_sem = pltpu.get_barrier_semaphore()
        for nbr in (left, right):
            pl.semaphore_signal(barrier_sem, inc=1, device_id=(nbr,),
                                device_id_type=pl.DeviceIdType.MESH)

        rows0 = pl.ds(0, mch)
        ld0[0][0].wait()
        OA[rows0, :] = SA[rows0, :].astype(jnp.bfloat16)
        ld0[0][1].wait()
        OB[rows0, :] = SB[rows0, :].astype(jnp.bfloat16)
        pl.semaphore_wait(barrier_sem, 2)

        rs = {}
        ag = {}
        for h in range(N_DEV - 1):
            for j in range(2):
                rows = pl.ds(j * mch, mch)
                rs["a", h, j] = pltpu.make_async_remote_copy(
                    src_ref=(OA.at[rows, :] if h == 0
                             else RA.at[(h - 1) % 2, rows, :]),
                    dst_ref=RA.at[h % 2, rows, :],
                    send_sem=rsa_s.at[h, j], recv_sem=rsa_r.at[h, j],
                    device_id=(right,), device_id_type=pl.DeviceIdType.MESH)
                rs["b", h, j] = pltpu.make_async_remote_copy(
                    src_ref=(OB.at[rows, :] if h == 0
                             else RB.at[(h - 1) % 2, rows, :]),
                    dst_ref=RB.at[h % 2, rows, :],
                    send_sem=rsb_s.at[h, j], recv_sem=rsb_r.at[h, j],
                    device_id=(left,), device_id_type=pl.DeviceIdType.MESH)
                ag["a", h, j] = pltpu.make_async_remote_copy(
                    src_ref=(OA.at[rows, :] if h == 0
                             else RA.at[h % 2, rows, :]),
                    dst_ref=RA.at[(h + 1) % 2, rows, :],
                    send_sem=aga_s.at[h, j], recv_sem=aga_r.at[h, j],
                    device_id=(right,), device_id_type=pl.DeviceIdType.MESH)
                ag["b", h, j] = pltpu.make_async_remote_copy(
                    src_ref=(OB.at[rows, :] if h == 0
                             else RB.at[h % 2, rows, :]),
                    dst_ref=RB.at[(h + 1) % 2, rows, :],
                    send_sem=agb_s.at[h, j], recv_sem=agb_r.at[h, j],
                    device_id=(left,), device_id_type=pl.DeviceIdType.MESH)

        def matmul_rows(dst, j, c0, c1):
            rows = pl.ds(j * mch, mch)
            dst[rows, :] = (
                jnp.dot(RA[0, rows, :], w_ref[pl.ds(0, kh), pl.ds(c0, nh)],
                        preferred_element_type=jnp.float32)
                + jnp.dot(RB[0, rows, :], w_ref[pl.ds(kh, kh), pl.ds(c0, nh)],
                          preferred_element_type=jnp.float32)
            ).astype(jnp.bfloat16)

        pending = []

        rs["a", 0, 0].start()
        rs["b", 0, 0].start()
        rows1 = pl.ds(mch, mch)
        ld0[1][0].wait()
        OA[rows1, :] = SA[rows1, :].astype(jnp.bfloat16)
        ld0[1][1].wait()
        OB[rows1, :] = SB[rows1, :].astype(jnp.bfloat16)
        rs["a", 0, 1].start()
        rs["b", 0, 1].start()
        ld_sa = pltpu.make_async_copy(
            t_hbm.at[row(my + 2), pl.ds(0, kh)], SA, sdma.at[2])
        ld_sb = pltpu.make_async_copy(
            t_hbm.at[row(my + 2), pl.ds(kh, kh)], SB, sdma.at[3])
        ld_sa.start()
        ld_sb.start()
        for h in range(N_DEV - 1):
            s = h % 2
            last = h == N_DEV - 2
            ld_sa.wait()
            ld_sb.wait()
            for j in range(2):
                rows = pl.ds(j * mch, mch)
                rs["a", h, j].wait()
                RA[s, rows, :] = (RA[s, rows, :]
                                  + SA[rows, :].astype(jnp.bfloat16))
                if not last:
                    rs["a", h + 1, j].start()
                rs["b", h, j].wait()
                RB[s, rows, :] = (RB[s, rows, :]
                                  + SB[rows, :].astype(jnp.bfloat16))
                if not last:
                    rs["b", h + 1, j].start()
                if last:
                    matmul_rows(OA, j, 0, nh)
                    ag["a", 0, j].start()
                    matmul_rows(OB, j, nh, n)
                    ag["b", 0, j].start()
                    st_a = pltpu.make_async_copy(
                        OA.at[rows, :], out_hbm.at[rowsub(my, j), pl.ds(0, nh)],
                        odma.at[12 + 2 * j])
                    st_b = pltpu.make_async_copy(
                        OB.at[rows, :], out_hbm.at[rowsub(my, j), pl.ds(nh, nh)],
                        odma.at[13 + 2 * j])
                    st_a.start()
                    st_b.start()
                    pending.append(st_a)
                    pending.append(st_b)
            if h < N_DEV - 2:
                ld_sa = pltpu.make_async_copy(
                    t_hbm.at[row(my + 2 - (h + 1)), pl.ds(0, kh)],
                    SA, sdma.at[2])
                ld_sb = pltpu.make_async_copy(
                    t_hbm.at[row(my + 2 + (h + 1)), pl.ds(kh, kh)],
                    SB, sdma.at[3])
                ld_sa.start()
                ld_sb.start()

        for h in range(N_DEV - 1):
            recv_slot = (h + 1) % 2
            for j in range(2):
                rows = pl.ds(j * mch, mch)
                ag["a", h, j].wait()
                if h < N_DEV - 2:
                    ag["a", h + 1, j].start()
                st_a = pltpu.make_async_copy(
                    RA.at[recv_slot, rows, :],
                    out_hbm.at[rowsub(my - 1 - h + N_DEV, j), pl.ds(0, nh)],
                    odma.at[4 * h + 2 * j])
                st_a.start()
                pending.append(st_a)
                ag["b", h, j].wait()
                if h < N_DEV - 2:
                    ag["b", h + 1, j].start()
                st_b = pltpu.make_async_copy(
                    RB.at[recv_slot, rows, :],
                    out_hbm.at[rowsub(my + 1 + h, j), pl.ds(nh, nh)],
                    odma.at[4 * h + 2 * j + 1])
                st_b.start()
                pending.append(st_b)

        for cp in pending:
            cp.wait()

    assert kh == nh

    return pl.pallas_call(
        body,
        out_shape=jax.ShapeDtypeStruct((m_shard, n), jnp.bfloat16),
        in_specs=[
            pl.BlockSpec(memory_space=pl.ANY),
            pl.BlockSpec(memory_space=pltpu.VMEM),
        ],
        out_specs=pl.BlockSpec(memory_space=pl.ANY),
        scratch_shapes=[
            pltpu.VMEM((mc, kh), jnp.float32),
            pltpu.VMEM((mc, kh), jnp.float32),
            pltpu.VMEM((2, mc, kh), jnp.bfloat16),
            pltpu.VMEM((2, mc, kh), jnp.bfloat16),
            pltpu.VMEM((mc, nh), jnp.bfloat16),
            pltpu.VMEM((mc, nh), jnp.bfloat16),
            pltpu.SemaphoreType.DMA((4,)),
            pltpu.SemaphoreType.DMA((16,)),
            pltpu.SemaphoreType.DMA((N_DEV - 1, 2)),
            pltpu.SemaphoreType.DMA((N_DEV - 1, 2)),
            pltpu.SemaphoreType.DMA((N_DEV - 1, 2)),
            pltpu.SemaphoreType.DMA((N_DEV - 1, 2)),
            pltpu.SemaphoreType.DMA((N_DEV - 1, 2)),
            pltpu.SemaphoreType.DMA((N_DEV - 1, 2)),
            pltpu.SemaphoreType.DMA((N_DEV - 1, 2)),
            pltpu.SemaphoreType.DMA((N_DEV - 1, 2)),
        ],
        compiler_params=pltpu.CompilerParams(
            collective_id=0, vmem_limit_bytes=63 * 1024 * 1024),
    )(t, W)
